# baseline (speedup 1.0000x reference)
import jax
import jax.numpy as jnp
from jax.experimental import pallas as pl
from jax.experimental.pallas import tpu as pltpu

_N, _D = 100000, 128
_CHUNKS = (160, 240, 400, 800, 1600, 3200, 6400,
           12400, 12400, 12400, 12400, 12400, 12400,
           6400, 3200, 1600, 800, 400, 240, 160)
_OFFS = tuple(sum(_CHUNKS[:i]) for i in range(len(_CHUNKS)))
_NCH = len(_CHUNKS)
assert sum(_CHUNKS) == _N


def _dma_pipeline(x_ref, o_ref, buf, in_sems, out_sems):
    for i in range(_NCH):
        pltpu.make_async_copy(
            x_ref.at[pl.ds(_OFFS[i], _CHUNKS[i])],
            buf.at[pl.ds(_OFFS[i], _CHUNKS[i])],
            in_sems.at[i],
        ).start()
    for i in range(_NCH):
        pltpu.make_async_copy(
            x_ref.at[pl.ds(_OFFS[i], _CHUNKS[i])],
            buf.at[pl.ds(_OFFS[i], _CHUNKS[i])],
            in_sems.at[i],
        ).wait()
        pltpu.make_async_copy(
            buf.at[pl.ds(_OFFS[i], _CHUNKS[i])],
            o_ref.at[pl.ds(_OFFS[i], _CHUNKS[i])],
            out_sems.at[i],
        ).start()
    for i in range(_NCH):
        pltpu.make_async_copy(
            buf.at[pl.ds(_OFFS[i], _CHUNKS[i])],
            o_ref.at[pl.ds(_OFFS[i], _CHUNKS[i])],
            out_sems.at[i],
        ).wait()


def kernel(data_list):
    return pl.pallas_call(
        _dma_pipeline,
        in_specs=[pl.BlockSpec(memory_space=pltpu.MemorySpace.HBM)],
        out_specs=pl.BlockSpec(memory_space=pltpu.MemorySpace.HBM),
        out_shape=jax.ShapeDtypeStruct((_N, _D), jnp.float32),
        scratch_shapes=[
            pltpu.VMEM((_N, _D), jnp.float32),
            pltpu.SemaphoreType.DMA((_NCH,)),
            pltpu.SemaphoreType.DMA((_NCH,)),
        ],
        compiler_params=pltpu.CompilerParams(vmem_limit_bytes=60 * 2**20),
    )(data_list)


# speedup vs baseline: 1.0031x; 1.0031x over previous
import jax
import jax.numpy as jnp
from jax.experimental import pallas as pl

_N, _D = 100000, 128
_BLOCK = 20000


def _copy_body(x_ref, o_ref):
    o_ref[...] = x_ref[...]


def kernel(data_list):
    return pl.pallas_call(
        _copy_body,
        grid=(_N // _BLOCK,),
        in_specs=[pl.BlockSpec((_BLOCK, _D), lambda i: (i, 0))],
        out_specs=pl.BlockSpec((_BLOCK, _D), lambda i: (i, 0)),
        out_shape=jax.ShapeDtypeStruct((_N, _D), jnp.float32),
    )(data_list)
